# Initial kernel scaffold; baseline (speedup 1.0000x reference)
import jax
import jax.numpy as jnp
from jax.experimental import pallas as pl


def kernel(nodes, edge_index, edge_attr, edge_type_mask, g1_Wm1, g1_bm1, g1_Wm2, g1_bm2, g1_Wu, g1_bu, g2_Wm1, g2_bm1, g2_Wm2, g2_bm2, g2_Wu, g2_bu, a_W1, a_b1, a_W2, a_b2):
    raise NotImplementedError("write your pallas kernel here")


# speedup vs baseline: 3.8987x; 3.8987x over previous
import functools

import jax
import jax.numpy as jnp
from jax import lax
from jax.experimental import pallas as pl
from jax.experimental.pallas import tpu as pltpu
from jax.experimental.pallas import tpu_sc as plsc

N = 10000
D = 128
E = 320000
ED = 16
H = 128
ESEL = 160000
HSEL = ESEL // 2

NPAD = 10240

NC, NS = 2, 16
NW = NC * NS
CH = 128
NCH_E = E // CH
NCH_A = HSEL // CH
NZR = NPAD // NS

def _cdiv(a, b):
  return (a + b - 1) // b


def _sc_mesh():
  return plsc.VectorSubcoreMesh(
      core_axis_name="c", subcore_axis_name="s", num_cores=NC, num_subcores=NS)


@functools.cache
def _sc_gather2_kernel():
  return pl.kernel(
      _sc_gather2_body,
      out_type=[jax.ShapeDtypeStruct((E, D), jnp.float32),
                jax.ShapeDtypeStruct((E, D), jnp.float32)],
      mesh=_sc_mesh(),
      scratch_types=[pltpu.VMEM((CH,), jnp.int32),
                     pltpu.VMEM((CH,), jnp.int32),
                     pltpu.VMEM((CH, D), jnp.float32),
                     pltpu.VMEM((CH, D), jnp.float32),
                     pltpu.SemaphoreType.DMA,
                     pltpu.SemaphoreType.DMA])


def _sc_gather2(*args):
  return _sc_gather2_kernel()(*args)


def _sc_gather2_body(p_hbm, q_hbm, ir_hbm, is_hbm, gr_hbm, gs_hbm,
                     ir_v, is_v, rr_v, rs_v, sem_r, sem_s):
  wid = lax.axis_index("s") * NC + lax.axis_index("c")

  def step(i, carry):
    c = i * NW + wid

    @pl.when(c < NCH_E)
    def _():
      base = c * CH
      pltpu.sync_copy(ir_hbm.at[pl.ds(base, CH)], ir_v)
      pltpu.sync_copy(is_hbm.at[pl.ds(base, CH)], is_v)
      cp1 = pltpu.async_copy(p_hbm.at[ir_v], rr_v, sem_r)
      cp2 = pltpu.async_copy(q_hbm.at[is_v], rs_v, sem_s)
      cp1.wait()
      cp2.wait()
      pltpu.sync_copy(rr_v, gr_hbm.at[pl.ds(base, CH)])
      pltpu.sync_copy(rs_v, gs_hbm.at[pl.ds(base, CH)])

    return carry

  lax.fori_loop(0, _cdiv(NCH_E, NW), step, 0)


@functools.cache
def _sc_scatter_add_kernel():
  return pl.kernel(
      _sc_scatter_add_body,
      out_type=jax.ShapeDtypeStruct((NC, NPAD, D), jnp.float32),
      mesh=_sc_mesh(),
      scratch_types=[pltpu.VMEM((CH,), jnp.int32),
                     pltpu.VMEM((CH, D), jnp.float32),
                     pltpu.VMEM_SHARED((NPAD, D), jnp.float32)])


def _sc_scatter_add(*args):
  return _sc_scatter_add_kernel()(*args)


def _sc_scatter_add_body(msg_hbm, ir_hbm, zeros_hbm, agg_hbm, idx_v, row_v,
                         acc_sh):
  cid = lax.axis_index("c")
  sid = lax.axis_index("s")
  wid = sid * NC + cid
  pltpu.sync_copy(zeros_hbm.at[pl.ds(sid * NZR, NZR)],
                  acc_sh.at[pl.ds(sid * NZR, NZR)])
  plsc.subcore_barrier()

  def step(i, carry):
    c = i * NW + wid

    @pl.when(c < NCH_E)
    def _():
      base = c * CH
      pltpu.sync_copy(ir_hbm.at[pl.ds(base, CH)], idx_v)
      pltpu.sync_copy(msg_hbm.at[pl.ds(base, CH)], row_v)
      pltpu.sync_copy(row_v, acc_sh.at[idx_v], add=True)

    return carry

  lax.fori_loop(0, _cdiv(NCH_E, NW), step, 0)
  plsc.subcore_barrier()
  pltpu.sync_copy(acc_sh.at[pl.ds(sid * NZR, NZR)],
                  agg_hbm.at[cid, pl.ds(sid * NZR, NZR)])


@functools.cache
def _sc_actor_gather_kernel():
  return pl.kernel(
      _sc_actor_gather_body,
      out_type=[jax.ShapeDtypeStruct((HSEL, D), jnp.float32),
                jax.ShapeDtypeStruct((HSEL, D), jnp.float32),
                jax.ShapeDtypeStruct((HSEL, D), jnp.float32)],
      mesh=_sc_mesh(),
      scratch_types=[pltpu.VMEM((CH,), jnp.int32),
                     pltpu.VMEM((CH,), jnp.int32),
                     pltpu.VMEM((CH,), jnp.int32),
                     pltpu.VMEM((CH, D), jnp.float32),
                     pltpu.VMEM((CH, D), jnp.float32),
                     pltpu.VMEM((CH, D), jnp.float32),
                     pltpu.SemaphoreType.DMA,
                     pltpu.SemaphoreType.DMA,
                     pltpu.SemaphoreType.DMA])


def _sc_actor_gather(*args):
  return _sc_actor_gather_kernel()(*args)


def _sc_actor_gather_body(qr_hbm, qs_hbm, ir_hbm, is_hbm, mask_hbm, ea_hbm,
                          ar_hbm, as_hbm, eao_hbm,
                          m_v, ir_v, is_v, rr_v, rs_v, ea_v, sem0, sem1, sem2):
  wid = lax.axis_index("s") * NC + lax.axis_index("c")

  def step(i, carry):
    c = i * NW + wid

    @pl.when(c < NCH_A)
    def _():
      base = c * CH
      pltpu.sync_copy(mask_hbm.at[pl.ds(base, CH)], m_v)
      cp0 = pltpu.async_copy(ir_hbm.at[m_v], ir_v, sem0)
      cp1 = pltpu.async_copy(is_hbm.at[m_v], is_v, sem1)
      cp2 = pltpu.async_copy(ea_hbm.at[m_v], ea_v, sem2)
      cp0.wait()
      cp1.wait()
      cp3 = pltpu.async_copy(qr_hbm.at[ir_v], rr_v, sem0)
      cp4 = pltpu.async_copy(qs_hbm.at[is_v], rs_v, sem1)
      cp2.wait()
      cp3.wait()
      cp4.wait()
      pltpu.sync_copy(rr_v, ar_hbm.at[pl.ds(base, CH)])
      pltpu.sync_copy(rs_v, as_hbm.at[pl.ds(base, CH)])
      pltpu.sync_copy(ea_v, eao_hbm.at[pl.ds(base, CH)])

    return carry

  lax.fori_loop(0, _cdiv(NCH_A, NW), step, 0)


RP = 2048
RM = 2000
RA = 2000


def _dual_proj_body(x_ref, wr_ref, ws_ref, p_ref, q_ref):
  x = x_ref[...]
  p_ref[...] = jnp.dot(x, wr_ref[...], preferred_element_type=jnp.float32)
  q_ref[...] = jnp.dot(x, ws_ref[...], preferred_element_type=jnp.float32)


def _tc_dual_proj(x, wr, ws):
  return pl.pallas_call(
      _dual_proj_body,
      grid=(NPAD // RP,),
      in_specs=[pl.BlockSpec((RP, D), lambda i: (i, 0)),
                pl.BlockSpec((D, H), lambda i: (0, 0)),
                pl.BlockSpec((D, H), lambda i: (0, 0))],
      out_specs=[pl.BlockSpec((RP, H), lambda i: (i, 0)),
                 pl.BlockSpec((RP, H), lambda i: (i, 0))],
      out_shape=[jax.ShapeDtypeStruct((NPAD, H), jnp.float32),
                 jax.ShapeDtypeStruct((NPAD, H), jnp.float32)],
  )(x, wr, ws)


def _msg_body(gr_ref, gs_ref, ea_ref, we_ref, b1_ref, w2_ref, b2_ref, out_ref):
  pre = (gr_ref[...] + gs_ref[...]
         + jnp.dot(ea_ref[...], we_ref[...], preferred_element_type=jnp.float32)
         + b1_ref[...])
  out_ref[...] = (jnp.dot(jnp.maximum(pre, 0.0), w2_ref[...],
                          preferred_element_type=jnp.float32) + b2_ref[...])


def _tc_msg(gr, gs, ea, we, b1, w2, b2):
  return pl.pallas_call(
      _msg_body,
      grid=(E // RM,),
      in_specs=[pl.BlockSpec((RM, H), lambda i: (i, 0)),
                pl.BlockSpec((RM, H), lambda i: (i, 0)),
                pl.BlockSpec((RM, ED), lambda i: (i, 0)),
                pl.BlockSpec((ED, H), lambda i: (0, 0)),
                pl.BlockSpec((1, H), lambda i: (0, 0)),
                pl.BlockSpec((H, H), lambda i: (0, 0)),
                pl.BlockSpec((1, H), lambda i: (0, 0))],
      out_specs=pl.BlockSpec((RM, H), lambda i: (i, 0)),
      out_shape=jax.ShapeDtypeStruct((E, H), jnp.float32),
  )(gr, gs, ea, we, b1, w2, b2)


def _update_body(x_ref, a0_ref, a1_ref, wt_ref, wb_ref, bu_ref, out_ref):
  acc = jnp.dot(x_ref[...], wt_ref[...], preferred_element_type=jnp.float32)
  acc += jnp.dot(a0_ref[...] + a1_ref[...], wb_ref[...],
                 preferred_element_type=jnp.float32)
  out_ref[...] = jnp.maximum(acc + bu_ref[...], 0.0)


def _tc_node_update(x, a0, a1, wt, wb, bu):
  return pl.pallas_call(
      _update_body,
      grid=(NPAD // RP,),
      in_specs=[pl.BlockSpec((RP, D), lambda i: (i, 0)),
                pl.BlockSpec((RP, H), lambda i: (i, 0)),
                pl.BlockSpec((RP, H), lambda i: (i, 0)),
                pl.BlockSpec((D, H), lambda i: (0, 0)),
                pl.BlockSpec((H, H), lambda i: (0, 0)),
                pl.BlockSpec((1, H), lambda i: (0, 0))],
      out_specs=pl.BlockSpec((RP, H), lambda i: (i, 0)),
      out_shape=jax.ShapeDtypeStruct((NPAD, H), jnp.float32),
  )(x, a0, a1, wt, wb, bu)


def _ea_body(ea_ref, we_ref, b1_ref, out_ref):
  out_ref[...] = (jnp.dot(ea_ref[...], we_ref[...],
                          preferred_element_type=jnp.float32) + b1_ref[...])


def _tc_ea_proj(ea, we, b1):
  return pl.pallas_call(
      _ea_body,
      grid=(E // RM,),
      in_specs=[pl.BlockSpec((RM, ED), lambda i: (i, 0)),
                pl.BlockSpec((ED, H), lambda i: (0, 0)),
                pl.BlockSpec((1, H), lambda i: (0, 0))],
      out_specs=pl.BlockSpec((RM, H), lambda i: (i, 0)),
      out_shape=jax.ShapeDtypeStruct((E, H), jnp.float32),
  )(ea, we, b1)


def _actor_body(are_ref, ase_ref, eae_ref, aro_ref, aso_ref, eao_ref,
                w2_ref, b2_ref, out_ref):
  pre_e = are_ref[...] + ase_ref[...] + eae_ref[...]
  pre_o = aro_ref[...] + aso_ref[...] + eao_ref[...]
  s = jnp.maximum(pre_e, 0.0) + jnp.maximum(pre_o, 0.0)
  m = 0.5 * jnp.dot(s, w2_ref[...], preferred_element_type=jnp.float32) \
      + b2_ref[...]
  lane = lax.broadcasted_iota(jnp.int32, m.shape, 1)
  out_ref[...] = jnp.where(lane == 0, m,
                           jnp.exp(jnp.clip(m, -20.0, 2.0)))


def _tc_actor_head(are, ase, eae, aro, aso, eao, w2, b2):
  return pl.pallas_call(
      _actor_body,
      grid=(HSEL // RA,),
      in_specs=[pl.BlockSpec((RA, H), lambda i: (i, 0)),
                pl.BlockSpec((RA, H), lambda i: (i, 0)),
                pl.BlockSpec((RA, H), lambda i: (i, 0)),
                pl.BlockSpec((RA, H), lambda i: (i, 0)),
                pl.BlockSpec((RA, H), lambda i: (i, 0)),
                pl.BlockSpec((RA, H), lambda i: (i, 0)),
                pl.BlockSpec((H, 2), lambda i: (0, 0)),
                pl.BlockSpec((1, 2), lambda i: (0, 0))],
      out_specs=pl.BlockSpec((RA, 2), lambda i: (i, 0)),
      out_shape=jax.ShapeDtypeStruct((HSEL, 2), jnp.float32),
  )(are, ase, eae, aro, aso, eao, w2, b2)


def _encoder_layer(x, idx_r, idx_s, ea, wm1, bm1, wm2, bm2, wu, bu, zeros):
  din = wm1.shape[0] - ED
  p, q = _tc_dual_proj(x, wm1[:din // 2], wm1[din // 2:din])
  gr, gs = _sc_gather2(p, q, idx_r, idx_s)
  msg = _tc_msg(gr, gs, ea, wm1[din:], bm1.reshape(1, H), wm2,
                bm2.reshape(1, H))
  agg = _sc_scatter_add(msg, idx_r, zeros)
  dup = wu.shape[0] - H
  return _tc_node_update(x, agg[0], agg[1], wu[:dup], wu[dup:],
                         bu.reshape(1, H))


def kernel(nodes, edge_index, edge_attr, edge_type_mask,
           g1_Wm1, g1_bm1, g1_Wm2, g1_bm2, g1_Wu, g1_bu,
           g2_Wm1, g2_bm1, g2_Wm2, g2_bm2, g2_Wu, g2_bu,
           a_W1, a_b1, a_W2, a_b2):
  idx_r = edge_index[0].astype(jnp.int32)
  idx_s = edge_index[1].astype(jnp.int32)
  ea = edge_attr[0]
  mask = edge_type_mask.astype(jnp.int32)
  x = jnp.pad(nodes[0], ((0, NPAD - N), (0, 0)))
  zeros = jnp.zeros((NPAD, D), jnp.float32)

  h = _encoder_layer(x, idx_r, idx_s, ea, g1_Wm1, g1_bm1, g1_Wm2, g1_bm2,
                     g1_Wu, g1_bu, zeros)
  h = _encoder_layer(h, idx_r, idx_s, ea, g2_Wm1, g2_bm1, g2_Wm2, g2_bm2,
                     g2_Wu, g2_bu, zeros)

  qr, qs = _tc_dual_proj(h, a_W1[:H], a_W1[H:2 * H])
  ea_all = _tc_ea_proj(ea, a_W1[2 * H:], a_b1.reshape(1, H))
  mask_e = mask[0::2]
  mask_o = mask[1::2]
  are, ase, eae = _sc_actor_gather(qr, qs, idx_r, idx_s, mask_e, ea_all)
  aro, aso, eao = _sc_actor_gather(qr, qs, idx_r, idx_s, mask_o, ea_all)
  out2 = _tc_actor_head(are, ase, eae, aro, aso, eao, a_W2,
                        a_b2.reshape(1, 2))
  mean = out2[:, 0].reshape(1, HSEL)
  std = out2[:, 1].reshape(1, HSEL)
  return (mean, std)


# speedup vs baseline: 4.5962x; 1.1789x over previous
import functools

import jax
import jax.numpy as jnp
from jax import lax
from jax.experimental import pallas as pl
from jax.experimental.pallas import tpu as pltpu
from jax.experimental.pallas import tpu_sc as plsc

N = 10000
D = 128
E = 320000
ED = 16
H = 128
ESEL = 160000
HSEL = ESEL // 2

NPAD = 10240

NC, NS = 2, 16
NW = NC * NS
CH = 128
NCH_E = E // CH
NCH_A = HSEL // CH
NZR = NPAD // NS

def _cdiv(a, b):
  return (a + b - 1) // b


def _sc_mesh():
  return plsc.VectorSubcoreMesh(
      core_axis_name="c", subcore_axis_name="s", num_cores=NC, num_subcores=NS)


_KG = 2


@functools.cache
def _sc_gather2_kernel():
  per_set = [pltpu.VMEM((CH,), jnp.int32),
             pltpu.VMEM((CH,), jnp.int32),
             pltpu.VMEM((CH, D), jnp.float32),
             pltpu.VMEM((CH, D), jnp.float32),
             pltpu.SemaphoreType.DMA,
             pltpu.SemaphoreType.DMA]
  return pl.kernel(
      _sc_gather2_body,
      out_type=[jax.ShapeDtypeStruct((E, D), jnp.float32),
                jax.ShapeDtypeStruct((E, D), jnp.float32)],
      mesh=_sc_mesh(),
      scratch_types=per_set * _KG)


def _sc_gather2(*args):
  return _sc_gather2_kernel()(*args)


def _sc_gather2_body(p_hbm, q_hbm, ir_hbm, is_hbm, gr_hbm, gs_hbm, *scratch):
  sets = [scratch[i * 6:(i + 1) * 6] for i in range(_KG)]
  wid = lax.axis_index("s") * NC + lax.axis_index("c")

  def step(g, carry):
    cs = [(g * _KG + b) * NW + wid for b in range(_KG)]
    for b in range(_KG):
      ir_v, is_v, rr_v, rs_v, sem_r, sem_s = sets[b]

      @pl.when(cs[b] < NCH_E)
      def _(b=b, ir_v=ir_v, is_v=is_v, rr_v=rr_v, rs_v=rs_v,
            sem_r=sem_r, sem_s=sem_s):
        base = cs[b] * CH
        pltpu.sync_copy(ir_hbm.at[pl.ds(base, CH)], ir_v)
        pltpu.sync_copy(is_hbm.at[pl.ds(base, CH)], is_v)
        pltpu.async_copy(p_hbm.at[ir_v], rr_v, sem_r)
        pltpu.async_copy(q_hbm.at[is_v], rs_v, sem_s)

    for b in range(_KG):
      ir_v, is_v, rr_v, rs_v, sem_r, sem_s = sets[b]

      @pl.when(cs[b] < NCH_E)
      def _(b=b, ir_v=ir_v, is_v=is_v, rr_v=rr_v, rs_v=rs_v,
            sem_r=sem_r, sem_s=sem_s):
        base = cs[b] * CH
        pltpu.make_async_copy(p_hbm.at[ir_v], rr_v, sem_r).wait()
        pltpu.make_async_copy(q_hbm.at[is_v], rs_v, sem_s).wait()
        pltpu.sync_copy(rr_v, gr_hbm.at[pl.ds(base, CH)])
        pltpu.sync_copy(rs_v, gs_hbm.at[pl.ds(base, CH)])

    return carry

  lax.fori_loop(0, _cdiv(_cdiv(NCH_E, NW), _KG), step, 0)


_KS = 2


@functools.cache
def _sc_scatter_add_kernel():
  per_set = [pltpu.VMEM((CH,), jnp.int32),
             pltpu.VMEM((CH, D), jnp.float32),
             pltpu.SemaphoreType.DMA]
  return pl.kernel(
      _sc_scatter_add_body,
      out_type=jax.ShapeDtypeStruct((NC, NPAD, D), jnp.float32),
      mesh=_sc_mesh(),
      scratch_types=per_set * _KS + [pltpu.VMEM_SHARED((NPAD, D), jnp.float32)])


def _sc_scatter_add(*args):
  return _sc_scatter_add_kernel()(*args)


def _sc_scatter_add_body(msg_hbm, ir_hbm, zeros_hbm, agg_hbm, *scratch):
  sets = [scratch[i * 3:(i + 1) * 3] for i in range(_KS)]
  acc_sh = scratch[-1]
  cid = lax.axis_index("c")
  sid = lax.axis_index("s")
  wid = sid * NC + cid
  pltpu.sync_copy(zeros_hbm.at[pl.ds(sid * NZR, NZR)],
                  acc_sh.at[pl.ds(sid * NZR, NZR)])
  plsc.subcore_barrier()

  def step(g, carry):
    cs = [(g * _KS + b) * NW + wid for b in range(_KS)]
    for b in range(_KS):
      idx_v, row_v, sem = sets[b]

      @pl.when(cs[b] < NCH_E)
      def _(b=b, idx_v=idx_v, row_v=row_v, sem=sem):
        base = cs[b] * CH
        pltpu.sync_copy(ir_hbm.at[pl.ds(base, CH)], idx_v)
        pltpu.async_copy(msg_hbm.at[pl.ds(base, CH)], row_v, sem)

    for b in range(_KS):
      idx_v, row_v, sem = sets[b]

      @pl.when(cs[b] < NCH_E)
      def _(b=b, idx_v=idx_v, row_v=row_v, sem=sem):
        base = cs[b] * CH
        pltpu.make_async_copy(msg_hbm.at[pl.ds(base, CH)], row_v, sem).wait()
        pltpu.sync_copy(row_v, acc_sh.at[idx_v], add=True)

    return carry

  lax.fori_loop(0, _cdiv(_cdiv(NCH_E, NW), _KS), step, 0)
  plsc.subcore_barrier()
  pltpu.sync_copy(acc_sh.at[pl.ds(sid * NZR, NZR)],
                  agg_hbm.at[cid, pl.ds(sid * NZR, NZR)])


_KA = 2


@functools.cache
def _sc_actor_gather_kernel():
  per_set = [pltpu.VMEM((CH,), jnp.int32),
             pltpu.VMEM((CH,), jnp.int32),
             pltpu.VMEM((CH,), jnp.int32),
             pltpu.VMEM((CH, D), jnp.float32),
             pltpu.VMEM((CH, D), jnp.float32),
             pltpu.VMEM((CH, D), jnp.float32),
             pltpu.SemaphoreType.DMA,
             pltpu.SemaphoreType.DMA,
             pltpu.SemaphoreType.DMA]
  return pl.kernel(
      _sc_actor_gather_body,
      out_type=[jax.ShapeDtypeStruct((HSEL, D), jnp.float32),
                jax.ShapeDtypeStruct((HSEL, D), jnp.float32),
                jax.ShapeDtypeStruct((HSEL, D), jnp.float32)],
      mesh=_sc_mesh(),
      scratch_types=per_set * _KA)


def _sc_actor_gather(*args):
  return _sc_actor_gather_kernel()(*args)


def _sc_actor_gather_body(qr_hbm, qs_hbm, ir_hbm, is_hbm, mask_hbm, ea_hbm,
                          ar_hbm, as_hbm, eao_hbm, *scratch):
  sets = [scratch[i * 9:(i + 1) * 9] for i in range(_KA)]
  wid = lax.axis_index("s") * NC + lax.axis_index("c")

  def step(g, carry):
    cs = [(g * _KA + b) * NW + wid for b in range(_KA)]

    for b in range(_KA):
      m_v, ir_v, is_v, rr_v, rs_v, ea_v, sem0, sem1, sem2 = sets[b]

      @pl.when(cs[b] < NCH_A)
      def _(b=b, m_v=m_v, ir_v=ir_v, is_v=is_v, ea_v=ea_v,
            sem0=sem0, sem1=sem1, sem2=sem2):
        base = cs[b] * CH
        pltpu.sync_copy(mask_hbm.at[pl.ds(base, CH)], m_v)
        pltpu.async_copy(ir_hbm.at[m_v], ir_v, sem0)
        pltpu.async_copy(is_hbm.at[m_v], is_v, sem1)
        pltpu.async_copy(ea_hbm.at[m_v], ea_v, sem2)

    for b in range(_KA):
      m_v, ir_v, is_v, rr_v, rs_v, ea_v, sem0, sem1, sem2 = sets[b]

      @pl.when(cs[b] < NCH_A)
      def _(b=b, m_v=m_v, ir_v=ir_v, is_v=is_v, rr_v=rr_v, rs_v=rs_v,
            sem0=sem0, sem1=sem1):
        pltpu.make_async_copy(ir_hbm.at[m_v], ir_v, sem0).wait()
        pltpu.make_async_copy(is_hbm.at[m_v], is_v, sem1).wait()
        pltpu.async_copy(qr_hbm.at[ir_v], rr_v, sem0)
        pltpu.async_copy(qs_hbm.at[is_v], rs_v, sem1)

    for b in range(_KA):
      m_v, ir_v, is_v, rr_v, rs_v, ea_v, sem0, sem1, sem2 = sets[b]

      @pl.when(cs[b] < NCH_A)
      def _(b=b, m_v=m_v, ir_v=ir_v, is_v=is_v, rr_v=rr_v, rs_v=rs_v,
            ea_v=ea_v, sem0=sem0, sem1=sem1, sem2=sem2):
        base = cs[b] * CH
        pltpu.make_async_copy(qr_hbm.at[ir_v], rr_v, sem0).wait()
        pltpu.make_async_copy(qs_hbm.at[is_v], rs_v, sem1).wait()
        pltpu.make_async_copy(ea_hbm.at[m_v], ea_v, sem2).wait()
        pltpu.sync_copy(rr_v, ar_hbm.at[pl.ds(base, CH)])
        pltpu.sync_copy(rs_v, as_hbm.at[pl.ds(base, CH)])
        pltpu.sync_copy(ea_v, eao_hbm.at[pl.ds(base, CH)])

    return carry

  lax.fori_loop(0, _cdiv(_cdiv(NCH_A, NW), _KA), step, 0)


RP = 2048
RM = 2000
RA = 2000


def _dual_proj_body(x_ref, wr_ref, ws_ref, p_ref, q_ref):
  x = x_ref[...]
  p_ref[...] = jnp.dot(x, wr_ref[...], preferred_element_type=jnp.float32)
  q_ref[...] = jnp.dot(x, ws_ref[...], preferred_element_type=jnp.float32)


def _tc_dual_proj(x, wr, ws):
  return pl.pallas_call(
      _dual_proj_body,
      grid=(NPAD // RP,),
      in_specs=[pl.BlockSpec((RP, D), lambda i: (i, 0)),
                pl.BlockSpec((D, H), lambda i: (0, 0)),
                pl.BlockSpec((D, H), lambda i: (0, 0))],
      out_specs=[pl.BlockSpec((RP, H), lambda i: (i, 0)),
                 pl.BlockSpec((RP, H), lambda i: (i, 0))],
      out_shape=[jax.ShapeDtypeStruct((NPAD, H), jnp.float32),
                 jax.ShapeDtypeStruct((NPAD, H), jnp.float32)],
  )(x, wr, ws)


def _msg_body(gr_ref, gs_ref, ea_ref, we_ref, b1_ref, w2_ref, b2_ref, out_ref):
  pre = (gr_ref[...] + gs_ref[...]
         + jnp.dot(ea_ref[...], we_ref[...], preferred_element_type=jnp.float32)
         + b1_ref[...])
  out_ref[...] = (jnp.dot(jnp.maximum(pre, 0.0), w2_ref[...],
                          preferred_element_type=jnp.float32) + b2_ref[...])


def _tc_msg(gr, gs, ea, we, b1, w2, b2):
  return pl.pallas_call(
      _msg_body,
      grid=(E // RM,),
      in_specs=[pl.BlockSpec((RM, H), lambda i: (i, 0)),
                pl.BlockSpec((RM, H), lambda i: (i, 0)),
                pl.BlockSpec((RM, ED), lambda i: (i, 0)),
                pl.BlockSpec((ED, H), lambda i: (0, 0)),
                pl.BlockSpec((1, H), lambda i: (0, 0)),
                pl.BlockSpec((H, H), lambda i: (0, 0)),
                pl.BlockSpec((1, H), lambda i: (0, 0))],
      out_specs=pl.BlockSpec((RM, H), lambda i: (i, 0)),
      out_shape=jax.ShapeDtypeStruct((E, H), jnp.float32),
  )(gr, gs, ea, we, b1, w2, b2)


def _update_body(x_ref, a0_ref, a1_ref, wt_ref, wb_ref, bu_ref, out_ref):
  acc = jnp.dot(x_ref[...], wt_ref[...], preferred_element_type=jnp.float32)
  acc += jnp.dot(a0_ref[...] + a1_ref[...], wb_ref[...],
                 preferred_element_type=jnp.float32)
  out_ref[...] = jnp.maximum(acc + bu_ref[...], 0.0)


def _tc_node_update(x, a0, a1, wt, wb, bu):
  return pl.pallas_call(
      _update_body,
      grid=(NPAD // RP,),
      in_specs=[pl.BlockSpec((RP, D), lambda i: (i, 0)),
                pl.BlockSpec((RP, H), lambda i: (i, 0)),
                pl.BlockSpec((RP, H), lambda i: (i, 0)),
                pl.BlockSpec((D, H), lambda i: (0, 0)),
                pl.BlockSpec((H, H), lambda i: (0, 0)),
                pl.BlockSpec((1, H), lambda i: (0, 0))],
      out_specs=pl.BlockSpec((RP, H), lambda i: (i, 0)),
      out_shape=jax.ShapeDtypeStruct((NPAD, H), jnp.float32),
  )(x, a0, a1, wt, wb, bu)


def _ea_body(ea_ref, we_ref, b1_ref, out_ref):
  out_ref[...] = (jnp.dot(ea_ref[...], we_ref[...],
                          preferred_element_type=jnp.float32) + b1_ref[...])


def _tc_ea_proj(ea, we, b1):
  return pl.pallas_call(
      _ea_body,
      grid=(E // RM,),
      in_specs=[pl.BlockSpec((RM, ED), lambda i: (i, 0)),
                pl.BlockSpec((ED, H), lambda i: (0, 0)),
                pl.BlockSpec((1, H), lambda i: (0, 0))],
      out_specs=pl.BlockSpec((RM, H), lambda i: (i, 0)),
      out_shape=jax.ShapeDtypeStruct((E, H), jnp.float32),
  )(ea, we, b1)


def _actor_body(are_ref, ase_ref, eae_ref, aro_ref, aso_ref, eao_ref,
                w2_ref, b2_ref, out_ref):
  pre_e = are_ref[...] + ase_ref[...] + eae_ref[...]
  pre_o = aro_ref[...] + aso_ref[...] + eao_ref[...]
  s = jnp.maximum(pre_e, 0.0) + jnp.maximum(pre_o, 0.0)
  m = 0.5 * jnp.dot(s, w2_ref[...], preferred_element_type=jnp.float32) \
      + b2_ref[...]
  lane = lax.broadcasted_iota(jnp.int32, m.shape, 1)
  out_ref[...] = jnp.where(lane == 0, m,
                           jnp.exp(jnp.clip(m, -20.0, 2.0)))


def _tc_actor_head(are, ase, eae, aro, aso, eao, w2, b2):
  return pl.pallas_call(
      _actor_body,
      grid=(HSEL // RA,),
      in_specs=[pl.BlockSpec((RA, H), lambda i: (i, 0)),
                pl.BlockSpec((RA, H), lambda i: (i, 0)),
                pl.BlockSpec((RA, H), lambda i: (i, 0)),
                pl.BlockSpec((RA, H), lambda i: (i, 0)),
                pl.BlockSpec((RA, H), lambda i: (i, 0)),
                pl.BlockSpec((RA, H), lambda i: (i, 0)),
                pl.BlockSpec((H, 2), lambda i: (0, 0)),
                pl.BlockSpec((1, 2), lambda i: (0, 0))],
      out_specs=pl.BlockSpec((RA, 2), lambda i: (i, 0)),
      out_shape=jax.ShapeDtypeStruct((HSEL, 2), jnp.float32),
  )(are, ase, eae, aro, aso, eao, w2, b2)


def _encoder_layer(x, idx_r, idx_s, ea, wm1, bm1, wm2, bm2, wu, bu, zeros):
  din = wm1.shape[0] - ED
  p, q = _tc_dual_proj(x, wm1[:din // 2], wm1[din // 2:din])
  gr, gs = _sc_gather2(p, q, idx_r, idx_s)
  msg = _tc_msg(gr, gs, ea, wm1[din:], bm1.reshape(1, H), wm2,
                bm2.reshape(1, H))
  agg = _sc_scatter_add(msg, idx_r, zeros)
  dup = wu.shape[0] - H
  return _tc_node_update(x, agg[0], agg[1], wu[:dup], wu[dup:],
                         bu.reshape(1, H))


def kernel(nodes, edge_index, edge_attr, edge_type_mask,
           g1_Wm1, g1_bm1, g1_Wm2, g1_bm2, g1_Wu, g1_bu,
           g2_Wm1, g2_bm1, g2_Wm2, g2_bm2, g2_Wu, g2_bu,
           a_W1, a_b1, a_W2, a_b2):
  idx_r = edge_index[0].astype(jnp.int32)
  idx_s = edge_index[1].astype(jnp.int32)
  ea = edge_attr[0]
  mask = edge_type_mask.astype(jnp.int32)
  x = jnp.pad(nodes[0], ((0, NPAD - N), (0, 0)))
  zeros = jnp.zeros((NPAD, D), jnp.float32)

  h = _encoder_layer(x, idx_r, idx_s, ea, g1_Wm1, g1_bm1, g1_Wm2, g1_bm2,
                     g1_Wu, g1_bu, zeros)
  h = _encoder_layer(h, idx_r, idx_s, ea, g2_Wm1, g2_bm1, g2_Wm2, g2_bm2,
                     g2_Wu, g2_bu, zeros)

  qr, qs = _tc_dual_proj(h, a_W1[:H], a_W1[H:2 * H])
  ea_all = _tc_ea_proj(ea, a_W1[2 * H:], a_b1.reshape(1, H))
  mask_e = mask[0::2]
  mask_o = mask[1::2]
  are, ase, eae = _sc_actor_gather(qr, qs, idx_r, idx_s, mask_e, ea_all)
  aro, aso, eao = _sc_actor_gather(qr, qs, idx_r, idx_s, mask_o, ea_all)
  out2 = _tc_actor_head(are, ase, eae, aro, aso, eao, a_W2,
                        a_b2.reshape(1, 2))
  mean = out2[:, 0].reshape(1, HSEL)
  std = out2[:, 1].reshape(1, HSEL)
  return (mean, std)
